# baseline (speedup 1.0000x reference)
import functools

import jax
import jax.numpy as jnp
import numpy as np
from jax import lax
from jax.experimental import pallas as pl
from jax.experimental.pallas import tpu as pltpu
from jax.experimental.pallas import tpu_sc as plsc

INPUT_DIM = 2
LOG2_HASHMAP = 19
NUM_LEVELS = 16
F_PER_LEVEL = 2
START_RES = 16
B_SCALE = 1.447269237440378
NUM_VEC = 2 ** LOG2_HASHMAP
MASK19 = NUM_VEC - 1
PI2_I32 = np.int32(np.uint32(2654435761).view(np.int32))
RES = [int(B_SCALE ** i * START_RES) for i in range(NUM_LEVELS)]

NC = 2
NS = 16
NW = NC * NS

B = 262144
NF = NUM_LEVELS * F_PER_LEVEL
PTS_PER_W = B // NW
C = 512
NCHUNK = PTS_PER_W // C
NGRP = C // 16
PTILE = C // 128
LVL_F32 = NUM_VEC * F_PER_LEVEL
STAGE_F32 = LVL_F32 // NS

def _share(l):
    n = F_PER_LEVEL * (RES[l] + 1) ** 2
    return -(-n // (NS * 128)) * 128

N_CACHED = 7
SHARES = [_share(l) for l in range(N_CACHED)]
GRID_F32 = NS * max(SHARES)


def _encode_body(x_hbm, tab_hbm, out_hbm,
                 tabs_s, idx_v, rows_v, x_v, grid_v, obuf, sem):
    wid = lax.axis_index("s") * NC + lax.axis_index("c")
    sid = lax.axis_index("s")
    iota = lax.iota(jnp.int32, 16)
    f128 = (iota & 1) * 128

    def load_x(n, xsel):
        t0 = wid * (PTS_PER_W // 128) + n * PTILE
        pltpu.sync_copy(x_hbm.at[pl.ds(t0, PTILE)], x_v.at[xsel])

    def frac_coords(xsel, g, res_f):
        q = lax.div(g, 8)
        s = lax.rem(g, 8) * 16
        p0 = x_v[xsel, q, 0, pl.ds(s, 16)]
        p1 = x_v[xsel, q, 1, pl.ds(s, 16)]
        xr0 = p0 * res_f
        xr1 = p1 * res_f
        i0 = xr0.astype(jnp.int32)
        i1 = xr1.astype(jnp.int32)
        d0 = xr0 - i0.astype(jnp.float32)
        d1 = xr1 - i1.astype(jnp.float32)
        return i0, i1, d0, d1

    def gen_fire(l, n, sel):
        res_f = np.float32(RES[l])
        load_x(n, sel)

        def body(g, carry):
            i0, i1, _, _ = frac_coords(sel, g, res_f)
            b0 = i1 * PI2_I32
            b1 = b0 + PI2_I32
            a1 = i0 + 1
            h00 = (i0 ^ b0) & MASK19
            h01 = (i0 ^ b1) & MASK19
            h10 = (a1 ^ b0) & MASK19
            h11 = (a1 ^ b1) & MASK19
            for c, h in enumerate((h00, h01, h10, h11)):
                f0 = h + (h & -128)
                idx_v[sel, g, pl.ds(c * 32, 16)] = f0
                idx_v[sel, g, pl.ds(c * 32 + 16, 16)] = f0 + 128
            pltpu.make_async_copy(
                tabs_s.at[idx_v.at[sel, g]],
                rows_v.at[sel, pl.ds(g * 128, 128)],
                sem.at[sel],
            ).start()
            return carry

        lax.fori_loop(0, NGRP, body, 0)

    def drain(sel):
        def body(g, carry):
            pltpu.make_async_copy(
                tabs_s.at[idx_v.at[sel, g]],
                rows_v.at[sel, pl.ds(g * 128, 128)],
                sem.at[sel],
            ).wait()
            return carry

        lax.fori_loop(0, NGRP, body, 0)

    def interp(l, n, sel):
        res_f = np.float32(RES[l])

        def body(g, carry):
            _, _, d0, d1 = frac_coords(sel, g, res_f)
            rbase = g * 128
            v = []
            for c in range(4):
                v.append((rows_v[sel, pl.ds(rbase + c * 32, 16)],
                          rows_v[sel, pl.ds(rbase + c * 32 + 16, 16)]))
            qc = lax.div(g, 8)
            cb = lax.rem(g, 8) * 16
            for f in range(2):
                c0 = v[0][f] + d0 * (v[2][f] - v[0][f])
                c1 = v[1][f] + d0 * (v[3][f] - v[1][f])
                cf = c0 + d1 * (c1 - c0)
                obuf[f, qc, 0, pl.ds(cb, 16)] = cf
            return carry

        lax.fori_loop(0, NGRP, body, 0)

    def build_grid(l):
        res1 = np.int32(RES[l] + 1)
        S = SHARES[l]
        loff = np.int32(l << (LOG2_HASHMAP + 1))
        nrow = S // 128
        base = sid * S

        def row(r, carry):
            e0 = base + r * 128
            for k in range(8):
                jv = lax.div(e0 + k * 16 + iota, 2)
                a = lax.div(jv, res1)
                b = lax.rem(jv, res1)
                h = (a ^ (b * PI2_I32)) & MASK19
                off = (h + (h & -128)) + f128 + loff
                idx_v[0, r, pl.ds(k * 16, 16)] = off
            pltpu.make_async_copy(
                tab_hbm.at[idx_v.at[0, r]],
                grid_v.at[pl.ds(base + r * 128, 128)],
                sem.at[0],
            ).start()
            return carry

        lax.fori_loop(0, nrow, row, 0)

        def row_wait(r, carry):
            pltpu.make_async_copy(
                tab_hbm.at[idx_v.at[0, r]],
                grid_v.at[pl.ds(base + r * 128, 128)],
                sem.at[0],
            ).wait()
            return carry

        lax.fori_loop(0, nrow, row_wait, 0)
        pltpu.sync_copy(grid_v.at[pl.ds(base, S)], tabs_s.at[pl.ds(base, S)])
        plsc.subcore_barrier()
        pltpu.sync_copy(tabs_s.at[pl.ds(0, NS * S)], grid_v.at[pl.ds(0, NS * S)])
        plsc.subcore_barrier()

    def cached_level(l):
        res_f = np.float32(RES[l])
        res1 = RES[l] + 1
        offs = [2 * (di * res1 + dj) + f
                for di in range(2) for dj in range(2) for f in range(2)]

        def chunk_body(n, carry):
            load_x(n, 0)

            def body(g, carry2):
                i0, i1, d0, d1 = frac_coords(0, g, res_f)
                m2 = (i0 * np.int32(res1) + i1) * 2
                v = [plsc.load_gather(grid_v, [m2 + np.int32(o)])
                     for o in offs]
                qc = lax.div(g, 8)
                cb = lax.rem(g, 8) * 16
                for f in range(2):
                    c0 = v[0 + f] + d0 * (v[4 + f] - v[0 + f])
                    c1 = v[2 + f] + d0 * (v[6 + f] - v[2 + f])
                    cf = c0 + d1 * (c1 - c0)
                    obuf[f, qc, 0, pl.ds(cb, 16)] = cf
                return carry2

            lax.fori_loop(0, NGRP, body, 0)
            t0 = wid * (PTS_PER_W // 128) + n * PTILE
            for f in range(2):
                ff = 2 * l + f
                pltpu.sync_copy(
                    obuf.at[f],
                    out_hbm.at[ff // 8, pl.ds(t0, PTILE), pl.ds(ff % 8, 1)])
            return carry

        lax.fori_loop(0, NCHUNK, chunk_body, 0)

    for l in range(N_CACHED):
        build_grid(l)
        cached_level(l)

    for l in range(N_CACHED, NUM_LEVELS):
        pltpu.sync_copy(
            tab_hbm.at[pl.ds(l * LVL_F32 + sid * STAGE_F32, STAGE_F32)],
            tabs_s.at[pl.ds(sid * STAGE_F32, STAGE_F32)])
        plsc.subcore_barrier()

        gen_fire(l, 0, 0)

        def chunk_body(n, carry, l=l):
            sel = lax.rem(n, 2)
            nsel = 1 - sel

            @pl.when(n + 1 < NCHUNK)
            def _():
                gen_fire(l, n + 1, nsel)

            drain(sel)
            interp(l, n, sel)
            t0 = wid * (PTS_PER_W // 128) + n * PTILE
            for f in range(2):
                ff = 2 * l + f
                pltpu.sync_copy(
                    obuf.at[f],
                    out_hbm.at[ff // 8, pl.ds(t0, PTILE), pl.ds(ff % 8, 1)])
            return carry

        lax.fori_loop(0, NCHUNK, chunk_body, 0)
        plsc.subcore_barrier()


@functools.partial(
    pl.kernel,
    out_type=jax.ShapeDtypeStruct((NF // 8, B // 128, 8, 128), jnp.float32),
    mesh=plsc.VectorSubcoreMesh(
        core_axis_name="c", subcore_axis_name="s",
        num_cores=NC, num_subcores=NS),
    compiler_params=pltpu.CompilerParams(
        needs_layout_passes=False, use_tc_tiling_on_sc=False),
    scratch_types=[
        pltpu.VMEM_SHARED((LVL_F32,), jnp.float32),
        pltpu.VMEM((2, NGRP, 128), jnp.int32),
        pltpu.VMEM((2, C * 8), jnp.float32),
        pltpu.VMEM((2, PTILE, INPUT_DIM, 128), jnp.float32),
        pltpu.VMEM((GRID_F32,), jnp.float32),
        pltpu.VMEM((F_PER_LEVEL, PTILE, 1, 128), jnp.float32),
        pltpu.SemaphoreType.DMA((2,)),
    ],
)
def _encode(*refs):
    _encode_body(*refs)


def kernel(x, tables):
    xv = x.reshape(B // 128, 128, INPUT_DIM).transpose(0, 2, 1)
    tabv = (tables.reshape(NUM_LEVELS, NUM_VEC // 128, 128, F_PER_LEVEL)
            .transpose(0, 1, 3, 2)
            .reshape(NUM_LEVELS * NUM_VEC * F_PER_LEVEL))
    out4 = _encode(xv, tabv)
    return out4.transpose(1, 3, 0, 2).reshape(B, NF)
